# Initial kernel scaffold; baseline (speedup 1.0000x reference)
import jax
import jax.numpy as jnp
from jax.experimental import pallas as pl


def kernel(inputs, labels):
    raise NotImplementedError("write your pallas kernel here")


# speedup vs baseline: 1.4497x; 1.4497x over previous
import functools
import math

import jax
import jax.numpy as jnp
from jax import lax
from jax.experimental import pallas as pl
from jax.experimental.pallas import tpu as pltpu
from jax.experimental.pallas import tpu_sc as plsc

S = 64.0
M = 0.5
T0 = 1.0
ALPHA = 0.01
B = 1024
C = 100000
COS_M = math.cos(M)
SIN_M = math.sin(M)
SHIFT = 2.0 * S


_NC = 2
_NS = 16
_L = 16
_NW = _NC * _NS
_B_PER_W = B // _NW


def _sc_gather_body(flat_hbm, labels_hbm, out_hbm, lab_v, idx_v, vals_v, sem):
    wid = lax.axis_index("s") * _NC + lax.axis_index("c")
    base = wid * _B_PER_W
    pltpu.sync_copy(labels_hbm.at[pl.ds(base, _B_PER_W)], lab_v)
    for j0 in range(0, _B_PER_W, _L):
        lab = lab_v[pl.ds(j0, _L)]
        row = base + j0 + lax.iota(jnp.int32, _L)
        idx_v[pl.ds(j0, _L)] = row * C + lab
    pltpu.async_copy(flat_hbm.at[idx_v], vals_v, sem).wait()
    pltpu.sync_copy(vals_v, out_hbm.at[pl.ds(base, _B_PER_W)])


@jax.jit
def _sc_gather(flat_inputs, labels):
    fn = functools.partial(
        pl.kernel,
        mesh=plsc.VectorSubcoreMesh(core_axis_name="c", subcore_axis_name="s"),
        out_type=jax.ShapeDtypeStruct((B,), jnp.float32),
        scratch_types=[
            pltpu.VMEM((_B_PER_W,), jnp.int32),
            pltpu.VMEM((_B_PER_W,), jnp.int32),
            pltpu.VMEM((_B_PER_W,), jnp.float32),
            pltpu.SemaphoreType.DMA,
        ],
    )(_sc_gather_body)
    return fn(flat_inputs, labels)


_BM = 128
_BN = 4096
_RB = B // _BM
_CB = (C + _BN - 1) // _BN


def _sweep_body(x_ref, tl_ref, out_ref, acc_ref, t_ref):
    i = pl.program_id(0)
    j = pl.program_id(1)

    @pl.when(jnp.logical_and(i == 0, j == 0))
    def _():
        tsum = jnp.sum(tl_ref[...])
        t_ref[0] = tsum * (ALPHA / B) + (1.0 - ALPHA) * T0
        out_ref[0, 0] = 0.0

    t_new = t_ref[0]
    tlb = tl_ref[pl.ds(i * _BM, _BM), :]
    ctm = tlb * COS_M - jnp.sqrt(1.0 - tlb * tlb) * SIN_M

    x = x_ref[...]
    hard = x > ctm
    mod = jnp.where(hard, x * (t_new + x), x)
    term = jnp.exp(mod * S - SHIFT)
    colid = j * _BN + lax.broadcasted_iota(jnp.int32, (_BM, _BN), 1)
    term = jnp.where(colid < C, term, 0.0)
    part = jnp.sum(term.reshape(_BM, _BN // 128, 128), axis=1)

    @pl.when(j == 0)
    def _():
        acc_ref[...] = part

    @pl.when(j > 0)
    def _():
        acc_ref[...] = acc_ref[...] + part

    @pl.when(j == _CB - 1)
    def _():
        row_sum = jnp.sum(acc_ref[...], axis=1, keepdims=True)
        lab_hard = tlb > ctm
        lab_mod = jnp.where(lab_hard, tlb * (t_new + tlb), tlb)
        lab_term = jnp.exp(lab_mod * S - SHIFT)
        ctm_term = jnp.exp(ctm * S - SHIFT)
        row_sum = row_sum - lab_term + ctm_term
        lse = SHIFT + jnp.log(row_sum)
        nll = lse - S * ctm
        out_ref[0, 0] += jnp.sum(nll) * (1.0 / B)


@jax.jit
def _tc_sweep(inputs, target_logit):
    tl2 = target_logit.reshape(B, 1)
    out = pl.pallas_call(
        _sweep_body,
        grid=(_RB, _CB),
        in_specs=[
            pl.BlockSpec((_BM, _BN), lambda i, j: (i, j)),
            pl.BlockSpec((B, 1), lambda i, j: (0, 0)),
        ],
        out_specs=pl.BlockSpec(memory_space=pltpu.SMEM),
        out_shape=jax.ShapeDtypeStruct((1, 1), jnp.float32),
        scratch_shapes=[
            pltpu.VMEM((_BM, 128), jnp.float32),
            pltpu.SMEM((1,), jnp.float32),
        ],
        compiler_params=pltpu.CompilerParams(
            dimension_semantics=("arbitrary", "arbitrary"),
        ),
    )(inputs, tl2)
    return out[0, 0]


def kernel(inputs, labels):
    flat = inputs.reshape(-1)
    target_logit = _sc_gather(flat, labels)
    return _tc_sweep(inputs, target_logit)


# speedup vs baseline: 2.5779x; 1.7782x over previous
import functools
import math

import jax
import jax.numpy as jnp
from jax import lax
from jax.experimental import pallas as pl
from jax.experimental.pallas import tpu as pltpu
from jax.experimental.pallas import tpu_sc as plsc

S = 64.0
M = 0.5
T0 = 1.0
ALPHA = 0.01
B = 1024
C = 100000
COS_M = math.cos(M)
SIN_M = math.sin(M)
SHIFT = 2.0 * S


_NC = 2
_NS = 16
_L = 16
_NW = _NC * _NS
_B_PER_W = B // _NW


def _sc_gather_body(x_hbm, labels_hbm, out_hbm, lab_v, rows_v, vals_v, sem):
    wid = lax.axis_index("s") * _NC + lax.axis_index("c")
    base = wid * _B_PER_W
    pltpu.sync_copy(labels_hbm.at[pl.ds(base, _B_PER_W)], lab_v)
    iota16 = lax.iota(jnp.int32, _L)
    for j0 in range(0, _B_PER_W, _L):
        lab16 = lab_v[pl.ds(j0, _L)]
        cb16 = jnp.bitwise_and(lab16, -128)
        copies = []
        for k in range(_L):
            j = j0 + k
            row8 = base + (j // 8) * 8
            copies.append(
                pltpu.async_copy(
                    x_hbm.at[pl.ds(row8, 8), pl.ds(pl.multiple_of(cb16[k], 128), 128)],
                    rows_v.at[pl.ds(k * 8, 8), :],
                    sem,
                )
            )
        for cp in copies:
            cp.wait()
        lane16 = jnp.bitwise_and(lab16, 127)
        vals_res = jnp.zeros((_L,), jnp.float32)
        for k in range(_L):
            sub = (j0 + k) % 8
            lane_k = lane16[k]
            start_k = pl.multiple_of(jnp.bitwise_and(lane_k, -_L), _L)
            chunk = rows_v[k * 8 + sub, pl.ds(start_k, _L)]
            p_k = jnp.full((_L,), jnp.bitwise_and(lane_k, _L - 1), jnp.int32)
            v_vec = lax.gather(
                chunk,
                p_k[:, None],
                lax.GatherDimensionNumbers(
                    offset_dims=(), collapsed_slice_dims=(0,), start_index_map=(0,)
                ),
                slice_sizes=(1,),
                mode=lax.GatherScatterMode.PROMISE_IN_BOUNDS,
            )
            vals_res = jnp.where(iota16 == k, v_vec, vals_res)
        vals_v[pl.ds(j0, _L)] = vals_res
    pltpu.sync_copy(vals_v, out_hbm.at[pl.ds(base, _B_PER_W)])


@jax.jit
def _sc_gather(inputs, labels):
    fn = functools.partial(
        pl.kernel,
        mesh=plsc.VectorSubcoreMesh(core_axis_name="c", subcore_axis_name="s"),
        out_type=jax.ShapeDtypeStruct((B,), jnp.float32),
        scratch_types=[
            pltpu.VMEM((_B_PER_W,), jnp.int32),
            pltpu.VMEM((_L * 8, 128), jnp.float32),
            pltpu.VMEM((_B_PER_W,), jnp.float32),
            pltpu.SemaphoreType.DMA,
        ],
    )(_sc_gather_body)
    return fn(inputs, labels)


_BM = 128
_BN = 4096
_RB = B // _BM
_CB = (C + _BN - 1) // _BN


def _sweep_body(x_ref, tl_ref, out_ref, acc_ref, t_ref):
    i = pl.program_id(0)
    j = pl.program_id(1)

    @pl.when(jnp.logical_and(i == 0, j == 0))
    def _():
        tsum = jnp.sum(tl_ref[...])
        t_ref[0] = tsum * (ALPHA / B) + (1.0 - ALPHA) * T0
        out_ref[0, 0] = 0.0

    t_new = t_ref[0]
    tlb = tl_ref[pl.ds(i * _BM, _BM), :]
    ctm = tlb * COS_M - jnp.sqrt(1.0 - tlb * tlb) * SIN_M

    x = x_ref[...]
    hard = x > ctm
    mod = jnp.where(hard, x * (t_new + x), x)
    term = jnp.exp(mod * S - SHIFT)
    colid = j * _BN + lax.broadcasted_iota(jnp.int32, (_BM, _BN), 1)
    term = jnp.where(colid < C, term, 0.0)
    part = jnp.sum(term.reshape(_BM, _BN // 128, 128), axis=1)

    @pl.when(j == 0)
    def _():
        acc_ref[...] = part

    @pl.when(j > 0)
    def _():
        acc_ref[...] = acc_ref[...] + part

    @pl.when(j == _CB - 1)
    def _():
        row_sum = jnp.sum(acc_ref[...], axis=1, keepdims=True)
        lab_hard = tlb > ctm
        lab_mod = jnp.where(lab_hard, tlb * (t_new + tlb), tlb)
        lab_term = jnp.exp(lab_mod * S - SHIFT)
        ctm_term = jnp.exp(ctm * S - SHIFT)
        row_sum = row_sum - lab_term + ctm_term
        lse = SHIFT + jnp.log(row_sum)
        nll = lse - S * ctm
        out_ref[0, 0] += jnp.sum(nll) * (1.0 / B)


@jax.jit
def _tc_sweep(inputs, target_logit):
    tl2 = target_logit.reshape(B, 1)
    out = pl.pallas_call(
        _sweep_body,
        grid=(_RB, _CB),
        in_specs=[
            pl.BlockSpec((_BM, _BN), lambda i, j: (i, j)),
            pl.BlockSpec((B, 1), lambda i, j: (0, 0)),
        ],
        out_specs=pl.BlockSpec(memory_space=pltpu.SMEM),
        out_shape=jax.ShapeDtypeStruct((1, 1), jnp.float32),
        scratch_shapes=[
            pltpu.VMEM((_BM, 128), jnp.float32),
            pltpu.SMEM((1,), jnp.float32),
        ],
        compiler_params=pltpu.CompilerParams(
            dimension_semantics=("arbitrary", "arbitrary"),
        ),
    )(inputs, tl2)
    return out[0, 0]


def kernel(inputs, labels):
    target_logit = _sc_gather(inputs, labels)
    return _tc_sweep(inputs, target_logit)
